# Initial kernel scaffold; baseline (speedup 1.0000x reference)
import functools
import jax
import jax.numpy as jnp
from jax import lax
from jax.experimental import pallas as pl
from jax.experimental.pallas import tpu as pltpu
from jax.experimental.pallas import tpu_sc as plsc

N = 10000
E = 320000
D = 128

NC = 2
NS = 16
NW = NC * NS
EPW = E // NW
CH = 80
NCHUNK = EPW // CH
RPT = N // NS
RCH = 125

_SC_MESH = plsc.VectorSubcoreMesh(core_axis_name="c", subcore_axis_name="s")


@functools.partial(
    pl.kernel,
    out_type=jax.ShapeDtypeStruct((NW, N), jnp.float32),
    mesh=_SC_MESH,
    scratch_types=[
        pltpu.VMEM((N,), jnp.float32),
        pltpu.VMEM((2000,), jnp.int32),
        pltpu.VMEM((2000,), jnp.float32),
        pltpu.SemaphoreType.DMA,
    ],
)
def _deg_kernel(dst_hbm, ew_hbm, out_hbm, degbuf, dbuf, wbuf, sem):
    c = lax.axis_index("c")
    s = lax.axis_index("s")
    wid = c * NS + s

    def zero(i, _):
        degbuf[pl.ds(i * 16, 16)] = jnp.zeros((16,), jnp.float32)
        return 0

    lax.fori_loop(0, N // 16, zero, 0)

    ebase = wid * EPW

    def chunk(j, _):
        off = ebase + j * 2000
        cp1 = pltpu.async_copy(dst_hbm.at[pl.ds(off, 2000)], dbuf, sem)
        cp2 = pltpu.async_copy(ew_hbm.at[pl.ds(off, 2000)], wbuf, sem)
        cp1.wait()
        cp2.wait()

        def vec(i, _):
            idx = dbuf[pl.ds(i * 16, 16)]
            w = wbuf[pl.ds(i * 16, 16)]
            plsc.addupdate_scatter(degbuf, [idx], w)
            return 0

        lax.fori_loop(0, 2000 // 16, vec, 0)
        return 0

    lax.fori_loop(0, EPW // 2000, chunk, 0)
    pltpu.sync_copy(degbuf, out_hbm.at[wid])


@functools.partial(
    pl.kernel,
    out_type=jax.ShapeDtypeStruct((NC, N, D), jnp.float32),
    mesh=_SC_MESH,
    scratch_types=[
        pltpu.VMEM_SHARED((N, D), jnp.float32),
        pltpu.VMEM((RCH, D), jnp.float32),
        pltpu.VMEM((CH,), jnp.int32),
        pltpu.VMEM((CH,), jnp.int32),
        pltpu.VMEM((CH,), jnp.float32),
        pltpu.VMEM((CH, D), jnp.float32),
        pltpu.SemaphoreType.DMA,
    ],
)
def _msg_kernel(hs_hbm, src_hbm, dst_hbm, ew_hbm, out_hbm,
                acc, zbuf, sidx, didx, ewb, rows, sem):
    c = lax.axis_index("c")
    s = lax.axis_index("s")
    wid = c * NS + s

    def zrow(r, _):
        for cc in range(8):
            zbuf[r, pl.ds(cc * 16, 16)] = jnp.zeros((16,), jnp.float32)
        return 0

    lax.fori_loop(0, RCH, zrow, 0)
    for k in range(RPT // RCH):
        pltpu.sync_copy(zbuf, acc.at[pl.ds(s * RPT + k * RCH, RCH)])
    plsc.subcore_barrier()

    ebase = wid * EPW

    def chunk(j, _):
        off = ebase + j * CH
        cp1 = pltpu.async_copy(src_hbm.at[pl.ds(off, CH)], sidx, sem)
        cp2 = pltpu.async_copy(dst_hbm.at[pl.ds(off, CH)], didx, sem)
        cp3 = pltpu.async_copy(ew_hbm.at[pl.ds(off, CH)], ewb, sem)
        cp1.wait()
        cp2.wait()
        cp3.wait()
        pltpu.async_copy(hs_hbm.at[sidx], rows, sem).wait()

        def scale(e, _):
            w = ewb[e]
            for cc in range(8):
                rows[e, pl.ds(cc * 16, 16)] = rows[e, pl.ds(cc * 16, 16)] * w
            return 0

        lax.fori_loop(0, CH, scale, 0)
        pltpu.sync_copy(rows, acc.at[didx], add=True)
        return 0

    lax.fori_loop(0, NCHUNK, chunk, 0)
    plsc.subcore_barrier()

    for k in range(RPT // RCH):
        r0 = s * RPT + k * RCH
        pltpu.sync_copy(acc.at[pl.ds(r0, RCH)], out_hbm.at[c, pl.ds(r0, RCH)])


_RB = 1000


def _m1_body(x_ref, w_ref, degp_ref, hs_ref, dinv_ref):
    deg = jnp.sum(degp_ref[...], axis=0) + 1.0
    dinv = lax.rsqrt(deg)
    h = lax.dot_general(x_ref[...], w_ref[...],
                        (((1,), (1,)), ((), ())),
                        preferred_element_type=jnp.float32)
    hs_ref[...] = h * dinv[:, None]
    dinv_ref[...] = dinv[:, None]


def _tc_stage1(x, w1, degp):
    return pl.pallas_call(
        _m1_body,
        grid=(N // _RB,),
        in_specs=[
            pl.BlockSpec((_RB, D), lambda i: (i, 0)),
            pl.BlockSpec((D, D), lambda i: (0, 0)),
            pl.BlockSpec((NW, _RB), lambda i: (0, i)),
        ],
        out_specs=[
            pl.BlockSpec((_RB, D), lambda i: (i, 0)),
            pl.BlockSpec((_RB, 1), lambda i: (i, 0)),
        ],
        out_shape=[
            jax.ShapeDtypeStruct((N, D), jnp.float32),
            jax.ShapeDtypeStruct((N, 1), jnp.float32),
        ],
    )(x, w1, degp)


def _m2_body(accp_ref, hs_ref, dinv_ref, b_ref, w_ref, out_ref):
    dinv = dinv_ref[...]
    z = dinv * (accp_ref[0] + accp_ref[1] + hs_ref[...]) + b_ref[...]
    r = jnp.maximum(z, 0.0)
    h2 = lax.dot_general(r, w_ref[...],
                         (((1,), (1,)), ((), ())),
                         preferred_element_type=jnp.float32)
    out_ref[...] = h2 * dinv


def _tc_stage2(accp, hs, dinv, b1, w2):
    return pl.pallas_call(
        _m2_body,
        grid=(N // _RB,),
        in_specs=[
            pl.BlockSpec((NC, _RB, D), lambda i: (0, i, 0)),
            pl.BlockSpec((_RB, D), lambda i: (i, 0)),
            pl.BlockSpec((_RB, 1), lambda i: (i, 0)),
            pl.BlockSpec((1, D), lambda i: (0, 0)),
            pl.BlockSpec((D, D), lambda i: (0, 0)),
        ],
        out_specs=pl.BlockSpec((_RB, D), lambda i: (i, 0)),
        out_shape=jax.ShapeDtypeStruct((N, D), jnp.float32),
    )(accp, hs, dinv, b1, w2)


def _m3_body(accp_ref, hs_ref, dinv_ref, b_ref, out_ref):
    dinv = dinv_ref[...]
    out_ref[...] = dinv * (accp_ref[0] + accp_ref[1] + hs_ref[...]) + b_ref[...]


def _tc_stage3(accp, hs, dinv, b2):
    return pl.pallas_call(
        _m3_body,
        grid=(N // _RB,),
        in_specs=[
            pl.BlockSpec((NC, _RB, D), lambda i: (0, i, 0)),
            pl.BlockSpec((_RB, D), lambda i: (i, 0)),
            pl.BlockSpec((_RB, 1), lambda i: (i, 0)),
            pl.BlockSpec((1, D), lambda i: (0, 0)),
        ],
        out_specs=pl.BlockSpec((_RB, D), lambda i: (i, 0)),
        out_shape=jax.ShapeDtypeStruct((N, D), jnp.float32),
    )(accp, hs, dinv, b2)


def kernel(x, edge_index, edge_weight, W1, b1, W2, b2):
    src = edge_index[0]
    dst = edge_index[1]
    b1r = b1.reshape(1, D)
    b2r = b2.reshape(1, D)

    degp = _deg_kernel(dst, edge_weight)
    hs1, dinv = _tc_stage1(x, W1, degp)
    acc1 = _msg_kernel(hs1, src, dst, edge_weight)
    hs2 = _tc_stage2(acc1, hs1, dinv, b1r, W2)
    acc2 = _msg_kernel(hs2, src, dst, edge_weight)
    out = _tc_stage3(acc2, hs2, dinv, b2r)
    return out


# speedup vs baseline: 7.7339x; 7.7339x over previous
import functools
import jax
import jax.numpy as jnp
from jax import lax
from jax.experimental import pallas as pl
from jax.experimental.pallas import tpu as pltpu
from jax.experimental.pallas import tpu_sc as plsc

N = 10000
E = 320000
D = 128

NC = 2
NS = 16
NW = NC * NS
NPAD = 10240
EPW = 10240
EP = NW * EPW
CH = 128
NCHUNK = EPW // CH
RPT = NPAD // NS
RCH = 128

_SC_MESH = plsc.VectorSubcoreMesh(core_axis_name="c", subcore_axis_name="s")


@functools.partial(
    pl.kernel,
    out_type=jax.ShapeDtypeStruct((NC, 1, NPAD), jnp.float32),
    mesh=_SC_MESH,
    scratch_types=[
        pltpu.VMEM_SHARED((NPAD,), jnp.float32),
        pltpu.VMEM((2048,), jnp.float32),
        pltpu.VMEM((CH,), jnp.int32),
        pltpu.VMEM((CH,), jnp.float32),
        pltpu.SemaphoreType.DMA,
    ],
)
def _deg_kernel(dst_hbm, ew_hbm, out_hbm, acc, zbuf, didx, wbuf, sem):
    c = lax.axis_index("c")
    s = lax.axis_index("s")
    wid = c * NS + s

    @pl.when(s == 0)
    def _():
        def z(i, _):
            zbuf[pl.ds(i * 16, 16)] = jnp.zeros((16,), jnp.float32)
            return 0

        lax.fori_loop(0, 2048 // 16, z, 0)
        for k in range(NPAD // 2048):
            pltpu.sync_copy(zbuf, acc.at[pl.ds(k * 2048, 2048)])

    plsc.subcore_barrier()

    ebase = wid * EPW

    def chunk(j, _):
        off = ebase + j * CH
        cp1 = pltpu.async_copy(dst_hbm.at[pl.ds(off, CH)], didx, sem)
        cp2 = pltpu.async_copy(ew_hbm.at[pl.ds(off, CH)], wbuf, sem)
        cp1.wait()
        cp2.wait()
        pltpu.sync_copy(wbuf, acc.at[didx], add=True)
        return 0

    lax.fori_loop(0, NCHUNK, chunk, 0)
    plsc.subcore_barrier()

    @pl.when(s == 0)
    def _():
        pltpu.sync_copy(acc, out_hbm.at[c, 0])


@functools.partial(
    pl.kernel,
    out_type=jax.ShapeDtypeStruct((NC, NPAD, D), jnp.float32),
    mesh=_SC_MESH,
    scratch_types=[
        pltpu.VMEM_SHARED((NPAD, D), jnp.float32),
        pltpu.VMEM((RCH, D), jnp.float32),
        pltpu.VMEM((CH,), jnp.int32),
        pltpu.VMEM((CH,), jnp.int32),
        pltpu.VMEM((CH,), jnp.float32),
        pltpu.VMEM((CH, D), jnp.float32),
        pltpu.SemaphoreType.DMA,
    ],
)
def _msg_kernel(hs_hbm, src_hbm, dst_hbm, ew_hbm, out_hbm,
                acc, zbuf, sidx, didx, ewb, rows, sem):
    c = lax.axis_index("c")
    s = lax.axis_index("s")
    wid = c * NS + s

    def zrow(r, _):
        for cc in range(8):
            zbuf[r, pl.ds(cc * 16, 16)] = jnp.zeros((16,), jnp.float32)
        return 0

    lax.fori_loop(0, RCH, zrow, 0)
    for k in range(RPT // RCH):
        pltpu.sync_copy(zbuf, acc.at[pl.ds(s * RPT + k * RCH, RCH)])
    plsc.subcore_barrier()

    ebase = wid * EPW

    def chunk(j, _):
        off = ebase + j * CH
        cp1 = pltpu.async_copy(src_hbm.at[pl.ds(off, CH)], sidx, sem)
        cp2 = pltpu.async_copy(dst_hbm.at[pl.ds(off, CH)], didx, sem)
        cp3 = pltpu.async_copy(ew_hbm.at[pl.ds(off, CH)], ewb, sem)
        cp1.wait()
        cp2.wait()
        cp3.wait()
        pltpu.async_copy(hs_hbm.at[sidx], rows, sem).wait()

        def scale(g, _):
            wv = ewb[pl.ds(g * 16, 16)]
            for l in range(16):
                w = wv[l]
                e = g * 16 + l
                for cc in range(8):
                    rows[e, pl.ds(cc * 16, 16)] = rows[e, pl.ds(cc * 16, 16)] * w
            return 0

        lax.fori_loop(0, CH // 16, scale, 0)
        pltpu.sync_copy(rows, acc.at[didx], add=True)
        return 0

    lax.fori_loop(0, NCHUNK, chunk, 0)
    plsc.subcore_barrier()

    for k in range(RPT // RCH):
        r0 = s * RPT + k * RCH
        pltpu.sync_copy(acc.at[pl.ds(r0, RCH)], out_hbm.at[c, pl.ds(r0, RCH)])


_RB = 1000


def _m1_body(x_ref, w_ref, degp_ref, hs_ref, dinv_ref):
    deg = jnp.sum(degp_ref[...], axis=1) + 1.0
    dinv = lax.rsqrt(deg)
    h = lax.dot_general(x_ref[...], w_ref[...],
                        (((1,), (1,)), ((), ())),
                        preferred_element_type=jnp.float32)
    hs_ref[...] = h * dinv[:, None]
    dinv_ref[...] = dinv[:, None]


def _tc_stage1(x, w1, degp):
    return pl.pallas_call(
        _m1_body,
        grid=(N // _RB,),
        in_specs=[
            pl.BlockSpec((_RB, D), lambda i: (i, 0)),
            pl.BlockSpec((D, D), lambda i: (0, 0)),
            pl.BlockSpec((_RB, NC), lambda i: (i, 0)),
        ],
        out_specs=[
            pl.BlockSpec((_RB, D), lambda i: (i, 0)),
            pl.BlockSpec((_RB, 1), lambda i: (i, 0)),
        ],
        out_shape=[
            jax.ShapeDtypeStruct((N, D), jnp.float32),
            jax.ShapeDtypeStruct((N, 1), jnp.float32),
        ],
    )(x, w1, degp)


def _m2_body(accp_ref, hs_ref, dinv_ref, b_ref, w_ref, out_ref):
    dinv = dinv_ref[...]
    z = dinv * (accp_ref[0] + accp_ref[1] + hs_ref[...]) + b_ref[...]
    r = jnp.maximum(z, 0.0)
    h2 = lax.dot_general(r, w_ref[...],
                         (((1,), (1,)), ((), ())),
                         preferred_element_type=jnp.float32)
    out_ref[...] = h2 * dinv


def _tc_stage2(accp, hs, dinv, b1, w2):
    return pl.pallas_call(
        _m2_body,
        grid=(N // _RB,),
        in_specs=[
            pl.BlockSpec((NC, _RB, D), lambda i: (0, i, 0)),
            pl.BlockSpec((_RB, D), lambda i: (i, 0)),
            pl.BlockSpec((_RB, 1), lambda i: (i, 0)),
            pl.BlockSpec((1, D), lambda i: (0, 0)),
            pl.BlockSpec((D, D), lambda i: (0, 0)),
        ],
        out_specs=pl.BlockSpec((_RB, D), lambda i: (i, 0)),
        out_shape=jax.ShapeDtypeStruct((N, D), jnp.float32),
    )(accp, hs, dinv, b1, w2)


def _m3_body(accp_ref, hs_ref, dinv_ref, b_ref, out_ref):
    dinv = dinv_ref[...]
    out_ref[...] = dinv * (accp_ref[0] + accp_ref[1] + hs_ref[...]) + b_ref[...]


def _tc_stage3(accp, hs, dinv, b2):
    return pl.pallas_call(
        _m3_body,
        grid=(N // _RB,),
        in_specs=[
            pl.BlockSpec((NC, _RB, D), lambda i: (0, i, 0)),
            pl.BlockSpec((_RB, D), lambda i: (i, 0)),
            pl.BlockSpec((_RB, 1), lambda i: (i, 0)),
            pl.BlockSpec((1, D), lambda i: (0, 0)),
        ],
        out_specs=pl.BlockSpec((_RB, D), lambda i: (i, 0)),
        out_shape=jax.ShapeDtypeStruct((N, D), jnp.float32),
    )(accp, hs, dinv, b2)


def kernel(x, edge_index, edge_weight, W1, b1, W2, b2):
    src = edge_index[0]
    dst = edge_index[1]
    pad = EP - E
    srcp = jnp.concatenate([src, jnp.zeros((pad,), src.dtype)])
    dstp = jnp.concatenate([dst, jnp.zeros((pad,), dst.dtype)])
    ewp = jnp.concatenate([edge_weight, jnp.zeros((pad,), edge_weight.dtype)])
    b1r = b1.reshape(1, D)
    b2r = b2.reshape(1, D)

    degp = _deg_kernel(dstp, ewp)
    degt = degp.reshape(NC, NPAD).T
    hs1, dinv = _tc_stage1(x, W1, degt)
    acc1 = _msg_kernel(hs1, srcp, dstp, ewp)
    hs2 = _tc_stage2(acc1, hs1, dinv, b1r, W2)
    acc2 = _msg_kernel(hs2, srcp, dstp, ewp)
    out = _tc_stage3(acc2, hs2, dinv, b2r)
    return out


# speedup vs baseline: 9.2160x; 1.1916x over previous
import functools
import jax
import jax.numpy as jnp
from jax import lax
from jax.experimental import pallas as pl
from jax.experimental.pallas import tpu as pltpu
from jax.experimental.pallas import tpu_sc as plsc

N = 10000
E = 320000
D = 128

NC = 2
NS = 16
NW = NC * NS
NPAD = 10240
EPW = 10240
EP = NW * EPW
CH = 128
NCHUNK = EPW // CH
RPT = NPAD // NS
RCH = 128
RZ = 64

_SC_MESH = plsc.VectorSubcoreMesh(core_axis_name="c", subcore_axis_name="s")


@functools.partial(
    pl.kernel,
    out_type=jax.ShapeDtypeStruct((NC, 1, NPAD), jnp.float32),
    mesh=_SC_MESH,
    scratch_types=[
        pltpu.VMEM_SHARED((NPAD,), jnp.float32),
        pltpu.VMEM((2048,), jnp.float32),
        pltpu.VMEM((CH,), jnp.int32),
        pltpu.VMEM((CH,), jnp.float32),
        pltpu.SemaphoreType.DMA,
    ],
)
def _deg_kernel(dst_hbm, ew_hbm, out_hbm, acc, zbuf, didx, wbuf, sem):
    c = lax.axis_index("c")
    s = lax.axis_index("s")
    wid = c * NS + s

    @pl.when(s == 0)
    def _():
        def z(i, _):
            zbuf[pl.ds(i * 16, 16)] = jnp.zeros((16,), jnp.float32)
            return 0

        lax.fori_loop(0, 2048 // 16, z, 0)
        for k in range(NPAD // 2048):
            pltpu.sync_copy(zbuf, acc.at[pl.ds(k * 2048, 2048)])

    plsc.subcore_barrier()

    ebase = wid * EPW

    def chunk(j, _):
        off = ebase + j * CH
        cp1 = pltpu.async_copy(dst_hbm.at[pl.ds(off, CH)], didx, sem)
        cp2 = pltpu.async_copy(ew_hbm.at[pl.ds(off, CH)], wbuf, sem)
        cp1.wait()
        cp2.wait()
        pltpu.sync_copy(wbuf, acc.at[didx], add=True)
        return 0

    lax.fori_loop(0, NCHUNK, chunk, 0)
    plsc.subcore_barrier()

    @pl.when(s == 0)
    def _():
        pltpu.sync_copy(acc, out_hbm.at[c, 0])


@functools.partial(
    pl.kernel,
    out_type=jax.ShapeDtypeStruct((NC, NPAD, D), jnp.float32),
    mesh=_SC_MESH,
    scratch_types=[
        pltpu.VMEM_SHARED((NPAD, D), jnp.float32),
        pltpu.VMEM((RZ, D), jnp.float32),
        pltpu.VMEM((2, CH), jnp.int32),
        pltpu.VMEM((2, CH), jnp.int32),
        pltpu.VMEM((2, CH), jnp.float32),
        pltpu.VMEM((2, CH, D), jnp.float32),
        pltpu.SemaphoreType.DMA,
        pltpu.SemaphoreType.DMA,
        pltpu.SemaphoreType.DMA,
        pltpu.SemaphoreType.DMA,
    ],
)
def _msg_kernel(hs_hbm, src_hbm, dst_hbm, ew_hbm, out_hbm,
                acc, zbuf, sidx, didx, ewb, rows, si0, si1, sr0, sr1):
    c = lax.axis_index("c")
    s = lax.axis_index("s")
    wid = c * NS + s
    semi = (si0, si1)
    semr = (sr0, sr1)

    def zrow(r, _):
        for cc in range(8):
            zbuf[r, pl.ds(cc * 16, 16)] = jnp.zeros((16,), jnp.float32)
        return 0

    lax.fori_loop(0, RZ, zrow, 0)
    for k in range(RPT // RZ):
        pltpu.sync_copy(zbuf, acc.at[pl.ds(s * RPT + k * RZ, RZ)])
    plsc.subcore_barrier()

    ebase = wid * EPW

    def fetch(j, b):
        off = ebase + j * CH
        pltpu.async_copy(src_hbm.at[pl.ds(off, CH)], sidx.at[b], semi[b])
        pltpu.async_copy(dst_hbm.at[pl.ds(off, CH)], didx.at[b], semi[b])
        pltpu.async_copy(ew_hbm.at[pl.ds(off, CH)], ewb.at[b], semi[b])

    def drain_idx(b):
        pltpu.make_async_copy(src_hbm.at[pl.ds(0, CH)], sidx.at[b], semi[b]).wait()
        pltpu.make_async_copy(dst_hbm.at[pl.ds(0, CH)], didx.at[b], semi[b]).wait()
        pltpu.make_async_copy(ew_hbm.at[pl.ds(0, CH)], ewb.at[b], semi[b]).wait()

    def start_gather(b):
        pltpu.async_copy(hs_hbm.at[sidx.at[b]], rows.at[b], semr[b])

    def drain_rows(b):
        pltpu.make_async_copy(hs_hbm.at[sidx.at[b]], rows.at[b], semr[b]).wait()

    fetch(0, 0)
    drain_idx(0)
    start_gather(0)
    fetch(1, 1)

    def body(jj, _):
        for b in (0, 1):
            j = jj * 2 + b
            drain_rows(b)

            @pl.when(j + 1 < NCHUNK)
            def _():
                drain_idx(1 - b)
                start_gather(1 - b)

            def scale(g, _):
                wv = ewb[b, pl.ds(g * 16, 16)]
                for l in range(16):
                    w = wv[l]
                    e = g * 16 + l
                    for cc in range(8):
                        rows[b, e, pl.ds(cc * 16, 16)] = (
                            rows[b, e, pl.ds(cc * 16, 16)] * w)
                return 0

            lax.fori_loop(0, CH // 16, scale, 0)
            pltpu.sync_copy(rows.at[b], acc.at[didx.at[b]], add=True)

            @pl.when(j + 2 < NCHUNK)
            def _():
                fetch(j + 2, b)
        return 0

    lax.fori_loop(0, NCHUNK // 2, body, 0)
    plsc.subcore_barrier()

    for k in range(RPT // RCH):
        r0 = s * RPT + k * RCH
        pltpu.sync_copy(acc.at[pl.ds(r0, RCH)], out_hbm.at[c, pl.ds(r0, RCH)])


_RB = 1000


def _m1_body(x_ref, w_ref, degp_ref, hs_ref, dinv_ref):
    deg = jnp.sum(degp_ref[...], axis=1) + 1.0
    dinv = lax.rsqrt(deg)
    h = lax.dot_general(x_ref[...], w_ref[...],
                        (((1,), (1,)), ((), ())),
                        preferred_element_type=jnp.float32)
    hs_ref[...] = h * dinv[:, None]
    dinv_ref[...] = dinv[:, None]


def _tc_stage1(x, w1, degp):
    return pl.pallas_call(
        _m1_body,
        grid=(N // _RB,),
        in_specs=[
            pl.BlockSpec((_RB, D), lambda i: (i, 0)),
            pl.BlockSpec((D, D), lambda i: (0, 0)),
            pl.BlockSpec((_RB, NC), lambda i: (i, 0)),
        ],
        out_specs=[
            pl.BlockSpec((_RB, D), lambda i: (i, 0)),
            pl.BlockSpec((_RB, 1), lambda i: (i, 0)),
        ],
        out_shape=[
            jax.ShapeDtypeStruct((N, D), jnp.float32),
            jax.ShapeDtypeStruct((N, 1), jnp.float32),
        ],
    )(x, w1, degp)


def _m2_body(accp_ref, hs_ref, dinv_ref, b_ref, w_ref, out_ref):
    dinv = dinv_ref[...]
    z = dinv * (accp_ref[0] + accp_ref[1] + hs_ref[...]) + b_ref[...]
    r = jnp.maximum(z, 0.0)
    h2 = lax.dot_general(r, w_ref[...],
                         (((1,), (1,)), ((), ())),
                         preferred_element_type=jnp.float32)
    out_ref[...] = h2 * dinv


def _tc_stage2(accp, hs, dinv, b1, w2):
    return pl.pallas_call(
        _m2_body,
        grid=(N // _RB,),
        in_specs=[
            pl.BlockSpec((NC, _RB, D), lambda i: (0, i, 0)),
            pl.BlockSpec((_RB, D), lambda i: (i, 0)),
            pl.BlockSpec((_RB, 1), lambda i: (i, 0)),
            pl.BlockSpec((1, D), lambda i: (0, 0)),
            pl.BlockSpec((D, D), lambda i: (0, 0)),
        ],
        out_specs=pl.BlockSpec((_RB, D), lambda i: (i, 0)),
        out_shape=jax.ShapeDtypeStruct((N, D), jnp.float32),
    )(accp, hs, dinv, b1, w2)


def _m3_body(accp_ref, hs_ref, dinv_ref, b_ref, out_ref):
    dinv = dinv_ref[...]
    out_ref[...] = dinv * (accp_ref[0] + accp_ref[1] + hs_ref[...]) + b_ref[...]


def _tc_stage3(accp, hs, dinv, b2):
    return pl.pallas_call(
        _m3_body,
        grid=(N // _RB,),
        in_specs=[
            pl.BlockSpec((NC, _RB, D), lambda i: (0, i, 0)),
            pl.BlockSpec((_RB, D), lambda i: (i, 0)),
            pl.BlockSpec((_RB, 1), lambda i: (i, 0)),
            pl.BlockSpec((1, D), lambda i: (0, 0)),
        ],
        out_specs=pl.BlockSpec((_RB, D), lambda i: (i, 0)),
        out_shape=jax.ShapeDtypeStruct((N, D), jnp.float32),
    )(accp, hs, dinv, b2)


def kernel(x, edge_index, edge_weight, W1, b1, W2, b2):
    src = edge_index[0]
    dst = edge_index[1]
    pad = EP - E
    srcp = jnp.concatenate([src, jnp.zeros((pad,), src.dtype)])
    dstp = jnp.concatenate([dst, jnp.zeros((pad,), dst.dtype)])
    ewp = jnp.concatenate([edge_weight, jnp.zeros((pad,), edge_weight.dtype)])
    b1r = b1.reshape(1, D)
    b2r = b2.reshape(1, D)

    degp = _deg_kernel(dstp, ewp)
    degt = degp.reshape(NC, NPAD).T
    hs1, dinv = _tc_stage1(x, W1, degt)
    acc1 = _msg_kernel(hs1, srcp, dstp, ewp)
    hs2 = _tc_stage2(acc1, hs1, dinv, b1r, W2)
    acc2 = _msg_kernel(hs2, srcp, dstp, ewp)
    out = _tc_stage3(acc2, hs2, dinv, b2r)
    return out
